# baseline (speedup 1.0000x reference)
import functools
import jax
import jax.numpy as jnp
from jax import lax
from jax.experimental import pallas as pl
from jax.experimental.pallas import tpu as pltpu
from jax.experimental.pallas import tpu_sc as plsc

N = 65536
HS = 128
NG = 64
NCELL = NG * NG
X_MIN, X_MAX = -3.0, 3.0
Y_MIN, Y_MAX = -3.0, 3.0
DX = (X_MAX - X_MIN) / NG
DY = (Y_MAX - Y_MIN) / NG

_TC_B = 2048
_NROW = N // HS
_RB = _TC_B // HS


def _lstm_tc_body(x_ref, h_ref, c_ref, xs_ref, ys_ref, wih_ref, whh_ref,
                  b_ref, hnew_ref, cnew_ref, cell_ref):
    xt = x_ref[...]
    h = h_ref[...].astype(jnp.bfloat16)
    c = c_ref[...]
    gates = (lax.dot_general(xt, wih_ref[...], (((0,), (0,)), ((), ())),
                             preferred_element_type=jnp.float32)
             + jnp.dot(h, whh_ref[...], preferred_element_type=jnp.float32)
             + b_ref[...])

    def sigmoid(z):
        return 0.5 * jnp.tanh(0.5 * z) + 0.5

    i = sigmoid(gates[:, 0:HS])
    f = sigmoid(gates[:, HS:2 * HS])
    g = jnp.tanh(gates[:, 2 * HS:3 * HS])
    o = sigmoid(gates[:, 3 * HS:4 * HS])
    c_new = f * c + i * g
    hnew_ref[...] = o * jnp.tanh(c_new)
    cnew_ref[...] = c_new
    xc = jnp.clip(xs_ref[...], X_MIN, X_MAX)
    yc = jnp.clip(ys_ref[...], Y_MIN, Y_MAX)
    xi = jnp.clip(jnp.floor((xc - X_MIN) / DX).astype(jnp.int32), 0, NG - 1)
    yi = jnp.clip(jnp.floor((yc - Y_MIN) / DY).astype(jnp.int32), 0, NG - 1)
    cell_ref[...] = xi * NG + yi


_NCHUNK = 4
_CB = N // _NCHUNK // _TC_B
_CROW = _NROW // _NCHUNK


def _lstm_tc_body2(x_ref, h_ref, c_ref, xs_ref, ys_ref, wih_ref, whh_ref,
                   b_ref, cdest_ref, hnew_ref, cnew_ref, cell_ref):
    del cdest_ref
    _lstm_tc_body(x_ref, h_ref, c_ref, xs_ref, ys_ref, wih_ref, whh_ref,
                  b_ref, hnew_ref, cnew_ref, cell_ref)


def _lstm_tc(k, coords_t, h, c, xs2d, ys2d, wih_t, whh_t, b2, c_donate=None,
             interpret=False):
    in_specs = [
        pl.BlockSpec((3, _TC_B), lambda i: (0, i + k * _CB)),
        pl.BlockSpec((_TC_B, HS), lambda i: (i + k * _CB, 0)),
        pl.BlockSpec((_TC_B, HS), lambda i: (i + k * _CB, 0)),
        pl.BlockSpec((_RB, HS), lambda i: (i + k * _CB, 0)),
        pl.BlockSpec((_RB, HS), lambda i: (i + k * _CB, 0)),
        pl.BlockSpec((3, 4 * HS), lambda i: (0, 0)),
        pl.BlockSpec((HS, 4 * HS), lambda i: (0, 0)),
        pl.BlockSpec((1, 4 * HS), lambda i: (0, 0)),
    ]
    args = [coords_t, h, c, xs2d, ys2d, wih_t, whh_t, b2]
    if c_donate is None:
        body = _lstm_tc_body
        aliases = {}
    else:
        body = _lstm_tc_body2
        in_specs = in_specs + [pl.BlockSpec((8, HS), lambda i: (0, 0))]
        args = args + [c_donate]
        aliases = {8: 1}
    return pl.pallas_call(
        body,
        grid=(_CB,),
        in_specs=in_specs,
        out_specs=[
            pl.BlockSpec((_TC_B, HS), lambda i: (i, 0)),
            pl.BlockSpec((_TC_B, HS), lambda i: (i + k * _CB, 0)),
            pl.BlockSpec((_RB, HS), lambda i: (i, 0)),
        ],
        out_shape=[
            jax.ShapeDtypeStruct((N // _NCHUNK, HS), jnp.float32),
            jax.ShapeDtypeStruct((N, HS), jnp.float32),
            jax.ShapeDtypeStruct((_CROW, HS), jnp.int32),
        ],
        input_output_aliases=aliases,
        interpret=interpret,
    )(*args)


@functools.cache
def _sc_mesh():
    return plsc.VectorSubcoreMesh(core_axis_name="c", subcore_axis_name="s",
                                  num_cores=2, num_subcores=16)
_RPC = _CROW // 2 // 16
_RPG = _NROW // 32


_NSLOT = 5


def _sc_scatter(h_new, cell2d, inits=None):
    init_args = [] if inits is None else list(inits)
    @functools.partial(
        pl.kernel,
        out_type=[jax.ShapeDtypeStruct((NCELL, HS), jnp.float32),
                  jax.ShapeDtypeStruct((NCELL, HS), jnp.float32)],
        mesh=_sc_mesh(),
        scratch_types=[
            pltpu.VMEM_SHARED((NCELL, HS), jnp.float32),
            pltpu.VMEM((_RPC, HS), jnp.int32),
            pltpu.VMEM((_NSLOT, HS, HS), jnp.float32),
        ] + [pltpu.SemaphoreType.DMA] * _NSLOT,
    )
    def scatter_k(h_hbm, cell_hbm, *rest):
        if inits is None:
            (t0_hbm, t1_hbm, tbl, idx_v, hbuf), sems = rest[:5], rest[5:]
        else:
            (i0_hbm, i1_hbm, t0_hbm, t1_hbm, tbl, idx_v, hbuf) = rest[:7]
            sems = rest[7:]
        c = lax.axis_index("c")
        s = lax.axis_index("s")
        if inits is None:
            @pl.loop(0, HS)
            def _(r):
                for cb in range(HS // 16):
                    hbuf.at[0, r, pl.ds(cb * 16, 16)][...] = jnp.zeros(
                        (16,), jnp.float32)

            pltpu.sync_copy(hbuf.at[0], tbl.at[pl.ds(s * 256, HS)])
            pltpu.sync_copy(hbuf.at[0], tbl.at[pl.ds(s * 256 + HS, HS)])
        else:
            @pl.when(c == 0)
            def _():
                pltpu.sync_copy(i0_hbm.at[pl.ds(s * 256, 256)],
                                tbl.at[pl.ds(s * 256, 256)])

            @pl.when(c == 1)
            def _():
                pltpu.sync_copy(i1_hbm.at[pl.ds(s * 256, 256)],
                                tbl.at[pl.ds(s * 256, 256)])
        plsc.subcore_barrier()
        row0 = c * (_CROW // 2) + s * _RPC
        pltpu.sync_copy(cell_hbm.at[pl.ds(row0, _RPC)], idx_v)

        def load(w, start):
            slot = w % _NSLOT
            args = (h_hbm.at[pl.ds((row0 + w) * HS, HS)], hbuf.at[slot],
                    sems[slot])
            if start:
                pltpu.async_copy(*args)
            else:
                pltpu.make_async_copy(*args).wait()

        def scat(w, start):
            slot = w % _NSLOT
            args = (hbuf.at[slot], tbl.at[idx_v.at[w]], sems[slot])
            if start:
                pltpu.async_copy(*args, add=True)
            else:
                pltpu.make_async_copy(*args).wait()

        for w in range(min(_NSLOT, _RPC)):
            load(w, True)
        for w in range(_RPC):
            load(w, False)
            scat(w, True)
            if w + _NSLOT < _RPC:
                scat(w, False)
                load(w + _NSLOT, True)
        for w in range(max(0, _RPC - _NSLOT), _RPC):
            scat(w, False)

        plsc.subcore_barrier()

        @pl.when(c == 0)
        def _():
            pltpu.sync_copy(tbl.at[pl.ds(s * 256, 256)],
                            t0_hbm.at[pl.ds(s * 256, 256)])

        @pl.when(c == 1)
        def _():
            pltpu.sync_copy(tbl.at[pl.ds(s * 256, 256)],
                            t1_hbm.at[pl.ds(s * 256, 256)])

    return scatter_k(h_new, cell2d, *init_args)


def _sc_gather(tabs, cell2d, iota2d):
    @functools.partial(
        pl.kernel,
        out_type=jax.ShapeDtypeStruct((N, HS), jnp.float32),
        mesh=_sc_mesh(),
        scratch_types=[
            pltpu.VMEM_SHARED((NCELL, HS), jnp.float32),
            pltpu.VMEM((2, HS), jnp.int32),
            pltpu.VMEM((_RPG, HS), jnp.int32),
            pltpu.VMEM((_NSLOT, HS, HS), jnp.float32),
        ] + [pltpu.SemaphoreType.DMA] * _NSLOT,
    )
    def gather_k(ta_hbm, tb_hbm, cell_hbm, iota_hbm, o_hbm,
                 tbl, iv, idx_v, buf, *sems):
        c = lax.axis_index("c")
        s = lax.axis_index("s")
        pltpu.sync_copy(iota_hbm.at[pl.ds(2 * s, 2)], iv)

        def stage(own_hbm, other_hbm):
            for h in range(2):
                pltpu.async_copy(
                    other_hbm.at[pl.ds(s * 256 + h * HS, HS)], buf.at[h],
                    sems[h])
            pltpu.sync_copy(own_hbm.at[pl.ds(s * 256, 256)],
                            tbl.at[pl.ds(s * 256, 256)])
            for h in range(2):
                pltpu.make_async_copy(
                    other_hbm.at[pl.ds(s * 256 + h * HS, HS)], buf.at[h],
                    sems[h]).wait()
                pltpu.async_copy(buf.at[h], tbl.at[iv.at[h]], sems[h],
                                 add=True)
            for h in range(2):
                pltpu.make_async_copy(buf.at[h], tbl.at[iv.at[h]],
                                      sems[h]).wait()

        @pl.when(c == 0)
        def _():
            stage(ta_hbm, tb_hbm)

        @pl.when(c == 1)
        def _():
            stage(tb_hbm, ta_hbm)
        plsc.subcore_barrier()
        row0 = (c * 16 + s) * _RPG
        pltpu.sync_copy(cell_hbm.at[pl.ds(row0, _RPG)], idx_v)

        def gath(w, start):
            slot = w % _NSLOT
            args = (tbl.at[idx_v.at[w]], buf.at[slot], sems[slot])
            if start:
                pltpu.async_copy(*args)
            else:
                pltpu.make_async_copy(*args).wait()

        def wout(w, start):
            slot = w % _NSLOT
            args = (buf.at[slot], o_hbm.at[pl.ds((row0 + w) * HS, HS)],
                    sems[slot])
            if start:
                pltpu.async_copy(*args)
            else:
                pltpu.make_async_copy(*args).wait()

        for w in range(min(_NSLOT, _RPG)):
            gath(w, True)
        for w in range(_RPG):
            gath(w, False)
            wout(w, True)
            if w + _NSLOT < _RPG:
                wout(w, False)
                gath(w + _NSLOT, True)
        for w in range(max(0, _RPG - _NSLOT), _RPG):
            wout(w, False)

    return gather_k(tabs[0], tabs[1], cell2d, iota2d)


def kernel(coords, hidden_state, cell_state, W_ih, W_hh, b_ih, b_hh):
    wih_t = W_ih.T.astype(jnp.bfloat16)
    whh_t = W_hh.T.astype(jnp.bfloat16)
    b2 = (b_ih + b_hh)[None, :]
    coords_t = coords.T.astype(jnp.bfloat16)
    xs2d = coords[:, 0].reshape(_NROW, HS)
    ys2d = coords[:, 1].reshape(_NROW, HS)
    hs, cds = [], []
    c_buf = None
    for k in range(_NCHUNK):
        hk, c_buf, cdk = _lstm_tc(k, coords_t, hidden_state, cell_state,
                                  xs2d, ys2d, wih_t, whh_t, b2,
                                  c_donate=c_buf)
        hs.append(hk)
        cds.append(cdk)
    iota2d = jnp.arange(NCELL, dtype=jnp.int32).reshape(NCELL // HS, HS)
    tabs = None
    for k in range(_NCHUNK):
        tabs = _sc_scatter(hs[k], cds[k], inits=tabs)
    cell2d = jnp.concatenate(cds, axis=0)
    h_social = _sc_gather(tabs, cell2d, iota2d)
    return (h_social, c_buf)


# speedup vs baseline: 1.0351x; 1.0351x over previous
import functools
import jax
import jax.numpy as jnp
from jax import lax
from jax.experimental import pallas as pl
from jax.experimental.pallas import tpu as pltpu
from jax.experimental.pallas import tpu_sc as plsc

N = 65536
HS = 128
NG = 64
NCELL = NG * NG
X_MIN, X_MAX = -3.0, 3.0
Y_MIN, Y_MAX = -3.0, 3.0
DX = (X_MAX - X_MIN) / NG
DY = (Y_MAX - Y_MIN) / NG

_TC_B = 2048
_NROW = N // HS
_RB = _TC_B // HS


def _lstm_tc_body(x_ref, h_ref, c_ref, xs_ref, ys_ref, wih_ref, whh_ref,
                  b_ref, hnew_ref, cnew_ref, cell_ref):
    xt = x_ref[...]
    h = h_ref[...].astype(jnp.bfloat16)
    c = c_ref[...]
    gates = (lax.dot_general(xt, wih_ref[...], (((0,), (0,)), ((), ())),
                             preferred_element_type=jnp.float32)
             + jnp.dot(h, whh_ref[...], preferred_element_type=jnp.float32)
             + b_ref[...])

    def sigmoid(z):
        return 0.5 * jnp.tanh(0.5 * z) + 0.5

    i = sigmoid(gates[:, 0:HS])
    f = sigmoid(gates[:, HS:2 * HS])
    g = jnp.tanh(gates[:, 2 * HS:3 * HS])
    o = sigmoid(gates[:, 3 * HS:4 * HS])
    c_new = f * c + i * g
    hnew_ref[...] = o * jnp.tanh(c_new)
    cnew_ref[...] = c_new
    xc = jnp.clip(xs_ref[...], X_MIN, X_MAX)
    yc = jnp.clip(ys_ref[...], Y_MIN, Y_MAX)
    xi = jnp.clip(jnp.floor((xc - X_MIN) / DX).astype(jnp.int32), 0, NG - 1)
    yi = jnp.clip(jnp.floor((yc - Y_MIN) / DY).astype(jnp.int32), 0, NG - 1)
    cell_ref[...] = xi * NG + yi


_NCHUNK = 2
_CB = N // _NCHUNK // _TC_B
_CROW = _NROW // _NCHUNK


def _lstm_tc_body2(x_ref, h_ref, c_ref, xs_ref, ys_ref, wih_ref, whh_ref,
                   b_ref, cdest_ref, hnew_ref, cnew_ref, cell_ref):
    del cdest_ref
    _lstm_tc_body(x_ref, h_ref, c_ref, xs_ref, ys_ref, wih_ref, whh_ref,
                  b_ref, hnew_ref, cnew_ref, cell_ref)


def _lstm_tc(k, coords_t, h, c, xs2d, ys2d, wih_t, whh_t, b2, c_donate=None,
             interpret=False):
    in_specs = [
        pl.BlockSpec((3, _TC_B), lambda i: (0, i + k * _CB)),
        pl.BlockSpec((_TC_B, HS), lambda i: (i + k * _CB, 0)),
        pl.BlockSpec((_TC_B, HS), lambda i: (i + k * _CB, 0)),
        pl.BlockSpec((_RB, HS), lambda i: (i + k * _CB, 0)),
        pl.BlockSpec((_RB, HS), lambda i: (i + k * _CB, 0)),
        pl.BlockSpec((3, 4 * HS), lambda i: (0, 0)),
        pl.BlockSpec((HS, 4 * HS), lambda i: (0, 0)),
        pl.BlockSpec((1, 4 * HS), lambda i: (0, 0)),
    ]
    args = [coords_t, h, c, xs2d, ys2d, wih_t, whh_t, b2]
    if c_donate is None:
        body = _lstm_tc_body
        aliases = {}
    else:
        body = _lstm_tc_body2
        in_specs = in_specs + [pl.BlockSpec((8, HS), lambda i: (0, 0))]
        args = args + [c_donate]
        aliases = {8: 1}
    return pl.pallas_call(
        body,
        grid=(_CB,),
        in_specs=in_specs,
        out_specs=[
            pl.BlockSpec((_TC_B, HS), lambda i: (i, 0)),
            pl.BlockSpec((_TC_B, HS), lambda i: (i + k * _CB, 0)),
            pl.BlockSpec((_RB, HS), lambda i: (i, 0)),
        ],
        out_shape=[
            jax.ShapeDtypeStruct((N // _NCHUNK, HS), jnp.float32),
            jax.ShapeDtypeStruct((N, HS), jnp.float32),
            jax.ShapeDtypeStruct((_CROW, HS), jnp.int32),
        ],
        input_output_aliases=aliases,
        interpret=interpret,
    )(*args)


@functools.cache
def _sc_mesh():
    return plsc.VectorSubcoreMesh(core_axis_name="c", subcore_axis_name="s",
                                  num_cores=2, num_subcores=16)
_RPC = _CROW // 2 // 16
_RPG = _NROW // 32


_NSLOT = 5


def _sc_scatter(h_new, cell2d, inits=None):
    init_args = [] if inits is None else list(inits)
    @functools.partial(
        pl.kernel,
        out_type=[jax.ShapeDtypeStruct((NCELL, HS), jnp.float32),
                  jax.ShapeDtypeStruct((NCELL, HS), jnp.float32)],
        mesh=_sc_mesh(),
        scratch_types=[
            pltpu.VMEM_SHARED((NCELL, HS), jnp.float32),
            pltpu.VMEM((_RPC, HS), jnp.int32),
            pltpu.VMEM((_NSLOT, HS, HS), jnp.float32),
        ] + [pltpu.SemaphoreType.DMA] * _NSLOT,
    )
    def scatter_k(h_hbm, cell_hbm, *rest):
        if inits is None:
            (t0_hbm, t1_hbm, tbl, idx_v, hbuf), sems = rest[:5], rest[5:]
        else:
            (i0_hbm, i1_hbm, t0_hbm, t1_hbm, tbl, idx_v, hbuf) = rest[:7]
            sems = rest[7:]
        c = lax.axis_index("c")
        s = lax.axis_index("s")
        if inits is None:
            @pl.loop(0, HS)
            def _(r):
                for cb in range(HS // 16):
                    hbuf.at[0, r, pl.ds(cb * 16, 16)][...] = jnp.zeros(
                        (16,), jnp.float32)

            pltpu.sync_copy(hbuf.at[0], tbl.at[pl.ds(s * 256, HS)])
            pltpu.sync_copy(hbuf.at[0], tbl.at[pl.ds(s * 256 + HS, HS)])
        else:
            @pl.when(c == 0)
            def _():
                pltpu.sync_copy(i0_hbm.at[pl.ds(s * 256, 256)],
                                tbl.at[pl.ds(s * 256, 256)])

            @pl.when(c == 1)
            def _():
                pltpu.sync_copy(i1_hbm.at[pl.ds(s * 256, 256)],
                                tbl.at[pl.ds(s * 256, 256)])
        plsc.subcore_barrier()
        row0 = c * (_CROW // 2) + s * _RPC
        pltpu.sync_copy(cell_hbm.at[pl.ds(row0, _RPC)], idx_v)

        def load(w, start):
            slot = w % _NSLOT
            args = (h_hbm.at[pl.ds((row0 + w) * HS, HS)], hbuf.at[slot],
                    sems[slot])
            if start:
                pltpu.async_copy(*args)
            else:
                pltpu.make_async_copy(*args).wait()

        def scat(w, start):
            slot = w % _NSLOT
            args = (hbuf.at[slot], tbl.at[idx_v.at[w]], sems[slot])
            if start:
                pltpu.async_copy(*args, add=True)
            else:
                pltpu.make_async_copy(*args).wait()

        for w in range(min(_NSLOT, _RPC)):
            load(w, True)
        for w in range(_RPC):
            load(w, False)
            scat(w, True)
            if w + _NSLOT < _RPC:
                scat(w, False)
                load(w + _NSLOT, True)
        for w in range(max(0, _RPC - _NSLOT), _RPC):
            scat(w, False)

        plsc.subcore_barrier()

        @pl.when(c == 0)
        def _():
            pltpu.sync_copy(tbl.at[pl.ds(s * 256, 256)],
                            t0_hbm.at[pl.ds(s * 256, 256)])

        @pl.when(c == 1)
        def _():
            pltpu.sync_copy(tbl.at[pl.ds(s * 256, 256)],
                            t1_hbm.at[pl.ds(s * 256, 256)])

    return scatter_k(h_new, cell2d, *init_args)


def _sc_gather(tabs, cell2d, iota2d):
    @functools.partial(
        pl.kernel,
        out_type=jax.ShapeDtypeStruct((N, HS), jnp.float32),
        mesh=_sc_mesh(),
        scratch_types=[
            pltpu.VMEM_SHARED((NCELL, HS), jnp.float32),
            pltpu.VMEM((2, HS), jnp.int32),
            pltpu.VMEM((_RPG, HS), jnp.int32),
            pltpu.VMEM((_NSLOT, HS, HS), jnp.float32),
        ] + [pltpu.SemaphoreType.DMA] * _NSLOT,
    )
    def gather_k(ta_hbm, tb_hbm, cell_hbm, iota_hbm, o_hbm,
                 tbl, iv, idx_v, buf, *sems):
        c = lax.axis_index("c")
        s = lax.axis_index("s")
        pltpu.sync_copy(iota_hbm.at[pl.ds(2 * s, 2)], iv)

        def stage(own_hbm, other_hbm):
            for h in range(2):
                pltpu.async_copy(
                    other_hbm.at[pl.ds(s * 256 + h * HS, HS)], buf.at[h],
                    sems[h])
            pltpu.sync_copy(own_hbm.at[pl.ds(s * 256, 256)],
                            tbl.at[pl.ds(s * 256, 256)])
            for h in range(2):
                pltpu.make_async_copy(
                    other_hbm.at[pl.ds(s * 256 + h * HS, HS)], buf.at[h],
                    sems[h]).wait()
                pltpu.async_copy(buf.at[h], tbl.at[iv.at[h]], sems[h],
                                 add=True)
            for h in range(2):
                pltpu.make_async_copy(buf.at[h], tbl.at[iv.at[h]],
                                      sems[h]).wait()

        @pl.when(c == 0)
        def _():
            stage(ta_hbm, tb_hbm)

        @pl.when(c == 1)
        def _():
            stage(tb_hbm, ta_hbm)
        plsc.subcore_barrier()
        row0 = (c * 16 + s) * _RPG
        pltpu.sync_copy(cell_hbm.at[pl.ds(row0, _RPG)], idx_v)

        def gath(w, start):
            slot = w % _NSLOT
            args = (tbl.at[idx_v.at[w]], buf.at[slot], sems[slot])
            if start:
                pltpu.async_copy(*args)
            else:
                pltpu.make_async_copy(*args).wait()

        def wout(w, start):
            slot = w % _NSLOT
            args = (buf.at[slot], o_hbm.at[pl.ds((row0 + w) * HS, HS)],
                    sems[slot])
            if start:
                pltpu.async_copy(*args)
            else:
                pltpu.make_async_copy(*args).wait()

        for w in range(min(_NSLOT, _RPG)):
            gath(w, True)
        for w in range(_RPG):
            gath(w, False)
            wout(w, True)
            if w + _NSLOT < _RPG:
                wout(w, False)
                gath(w + _NSLOT, True)
        for w in range(max(0, _RPG - _NSLOT), _RPG):
            wout(w, False)

    return gather_k(tabs[0], tabs[1], cell2d, iota2d)


def kernel(coords, hidden_state, cell_state, W_ih, W_hh, b_ih, b_hh):
    wih_t = W_ih.T.astype(jnp.bfloat16)
    whh_t = W_hh.T.astype(jnp.bfloat16)
    b2 = (b_ih + b_hh)[None, :]
    coords_t = coords.T.astype(jnp.bfloat16)
    xs2d = coords[:, 0].reshape(_NROW, HS)
    ys2d = coords[:, 1].reshape(_NROW, HS)
    hs, cds = [], []
    c_buf = None
    for k in range(_NCHUNK):
        hk, c_buf, cdk = _lstm_tc(k, coords_t, hidden_state, cell_state,
                                  xs2d, ys2d, wih_t, whh_t, b2,
                                  c_donate=c_buf)
        hs.append(hk)
        cds.append(cdk)
    iota2d = jnp.arange(NCELL, dtype=jnp.int32).reshape(NCELL // HS, HS)
    tabs = None
    for k in range(_NCHUNK):
        tabs = _sc_scatter(hs[k], cds[k], inits=tabs)
    cell2d = jnp.concatenate(cds, axis=0)
    h_social = _sc_gather(tabs, cell2d, iota2d)
    return (h_social, c_buf)


# speedup vs baseline: 1.0417x; 1.0064x over previous
import functools
import jax
import jax.numpy as jnp
from jax import lax
from jax.experimental import pallas as pl
from jax.experimental.pallas import tpu as pltpu
from jax.experimental.pallas import tpu_sc as plsc

N = 65536
HS = 128
NG = 64
NCELL = NG * NG
X_MIN, X_MAX = -3.0, 3.0
Y_MIN, Y_MAX = -3.0, 3.0
DX = (X_MAX - X_MIN) / NG
DY = (Y_MAX - Y_MIN) / NG

_TC_B = 2048
_NROW = N // HS
_RB = _TC_B // HS


def _lstm_tc_body(x_ref, h_ref, c_ref, xs_ref, ys_ref, wih_ref, whh_ref,
                  bi_ref, bh_ref, hnew_ref, cnew_ref, cell_ref):
    xt = x_ref[...]
    h = h_ref[...].astype(jnp.bfloat16)
    c = c_ref[...]
    gates = (lax.dot_general(xt, wih_ref[...], (((0,), (0,)), ((), ())),
                             preferred_element_type=jnp.float32)
             + jnp.dot(h, whh_ref[...], preferred_element_type=jnp.float32)
             + (bi_ref[...] + bh_ref[...]))

    def sigmoid(z):
        return 0.5 * jnp.tanh(0.5 * z) + 0.5

    i = sigmoid(gates[:, 0:HS])
    f = sigmoid(gates[:, HS:2 * HS])
    g = jnp.tanh(gates[:, 2 * HS:3 * HS])
    o = sigmoid(gates[:, 3 * HS:4 * HS])
    c_new = f * c + i * g
    hnew_ref[...] = o * jnp.tanh(c_new)
    cnew_ref[...] = c_new
    xc = jnp.clip(xs_ref[...], X_MIN, X_MAX)
    yc = jnp.clip(ys_ref[...], Y_MIN, Y_MAX)
    xi = jnp.clip(jnp.floor((xc - X_MIN) / DX).astype(jnp.int32), 0, NG - 1)
    yi = jnp.clip(jnp.floor((yc - Y_MIN) / DY).astype(jnp.int32), 0, NG - 1)
    cell_ref[...] = xi * NG + yi


_NCHUNK = 2
_CB = N // _NCHUNK // _TC_B
_CROW = _NROW // _NCHUNK


def _lstm_tc_body2(x_ref, h_ref, c_ref, xs_ref, ys_ref, wih_ref, whh_ref,
                   bi_ref, bh_ref, cdest_ref, hnew_ref, cnew_ref, cell_ref):
    del cdest_ref
    _lstm_tc_body(x_ref, h_ref, c_ref, xs_ref, ys_ref, wih_ref, whh_ref,
                  bi_ref, bh_ref, hnew_ref, cnew_ref, cell_ref)


def _lstm_tc(k, coords_t, h, c, xs2d, ys2d, wih_t, whh_t, b_i, b_h,
             c_donate=None, interpret=False):
    in_specs = [
        pl.BlockSpec((3, _TC_B), lambda i: (0, i + k * _CB)),
        pl.BlockSpec((_TC_B, HS), lambda i: (i + k * _CB, 0)),
        pl.BlockSpec((_TC_B, HS), lambda i: (i + k * _CB, 0)),
        pl.BlockSpec((_RB, HS), lambda i: (i + k * _CB, 0)),
        pl.BlockSpec((_RB, HS), lambda i: (i + k * _CB, 0)),
        pl.BlockSpec((3, 4 * HS), lambda i: (0, 0)),
        pl.BlockSpec((HS, 4 * HS), lambda i: (0, 0)),
        pl.BlockSpec((1, 4 * HS), lambda i: (0, 0)),
        pl.BlockSpec((1, 4 * HS), lambda i: (0, 0)),
    ]
    args = [coords_t, h, c, xs2d, ys2d, wih_t, whh_t, b_i, b_h]
    if c_donate is None:
        body = _lstm_tc_body
        aliases = {}
    else:
        body = _lstm_tc_body2
        in_specs = in_specs + [pl.BlockSpec((8, HS), lambda i: (0, 0))]
        args = args + [c_donate]
        aliases = {9: 1}
    return pl.pallas_call(
        body,
        grid=(_CB,),
        in_specs=in_specs,
        out_specs=[
            pl.BlockSpec((_TC_B, HS), lambda i: (i, 0)),
            pl.BlockSpec((_TC_B, HS), lambda i: (i + k * _CB, 0)),
            pl.BlockSpec((_RB, HS), lambda i: (i, 0)),
        ],
        out_shape=[
            jax.ShapeDtypeStruct((N // _NCHUNK, HS), jnp.float32),
            jax.ShapeDtypeStruct((N, HS), jnp.float32),
            jax.ShapeDtypeStruct((_CROW, HS), jnp.int32),
        ],
        input_output_aliases=aliases,
        interpret=interpret,
    )(*args)


@functools.cache
def _sc_mesh():
    return plsc.VectorSubcoreMesh(core_axis_name="c", subcore_axis_name="s",
                                  num_cores=2, num_subcores=16)
_RPC = _CROW // 2 // 16
_RPG = _NROW // 32


_NSLOT = 5


def _sc_scatter(h_new, cell2d, inits=None):
    init_args = [] if inits is None else list(inits)
    @functools.partial(
        pl.kernel,
        out_type=[jax.ShapeDtypeStruct((NCELL, HS), jnp.float32),
                  jax.ShapeDtypeStruct((NCELL, HS), jnp.float32)],
        mesh=_sc_mesh(),
        scratch_types=[
            pltpu.VMEM_SHARED((NCELL, HS), jnp.float32),
            pltpu.VMEM((_RPC, HS), jnp.int32),
            pltpu.VMEM((_NSLOT, HS, HS), jnp.float32),
        ] + [pltpu.SemaphoreType.DMA] * _NSLOT,
    )
    def scatter_k(h_hbm, cell_hbm, *rest):
        if inits is None:
            (t0_hbm, t1_hbm, tbl, idx_v, hbuf), sems = rest[:5], rest[5:]
        else:
            (i0_hbm, i1_hbm, t0_hbm, t1_hbm, tbl, idx_v, hbuf) = rest[:7]
            sems = rest[7:]
        c = lax.axis_index("c")
        s = lax.axis_index("s")
        if inits is None:
            @pl.loop(0, HS)
            def _(r):
                for cb in range(HS // 16):
                    hbuf.at[0, r, pl.ds(cb * 16, 16)][...] = jnp.zeros(
                        (16,), jnp.float32)

            pltpu.sync_copy(hbuf.at[0], tbl.at[pl.ds(s * 256, HS)])
            pltpu.sync_copy(hbuf.at[0], tbl.at[pl.ds(s * 256 + HS, HS)])
        else:
            @pl.when(c == 0)
            def _():
                pltpu.sync_copy(i0_hbm.at[pl.ds(s * 256, 256)],
                                tbl.at[pl.ds(s * 256, 256)])

            @pl.when(c == 1)
            def _():
                pltpu.sync_copy(i1_hbm.at[pl.ds(s * 256, 256)],
                                tbl.at[pl.ds(s * 256, 256)])
        plsc.subcore_barrier()
        row0 = c * (_CROW // 2) + s * _RPC
        pltpu.sync_copy(cell_hbm.at[pl.ds(row0, _RPC)], idx_v)

        def load(w, start):
            slot = w % _NSLOT
            args = (h_hbm.at[pl.ds((row0 + w) * HS, HS)], hbuf.at[slot],
                    sems[slot])
            if start:
                pltpu.async_copy(*args)
            else:
                pltpu.make_async_copy(*args).wait()

        def scat(w, start):
            slot = w % _NSLOT
            args = (hbuf.at[slot], tbl.at[idx_v.at[w]], sems[slot])
            if start:
                pltpu.async_copy(*args, add=True)
            else:
                pltpu.make_async_copy(*args).wait()

        for w in range(min(_NSLOT, _RPC)):
            load(w, True)
        for w in range(_RPC):
            load(w, False)
            scat(w, True)
            if w + _NSLOT < _RPC:
                scat(w, False)
                load(w + _NSLOT, True)
        for w in range(max(0, _RPC - _NSLOT), _RPC):
            scat(w, False)

        plsc.subcore_barrier()

        @pl.when(c == 0)
        def _():
            pltpu.sync_copy(tbl.at[pl.ds(s * 256, 256)],
                            t0_hbm.at[pl.ds(s * 256, 256)])

        @pl.when(c == 1)
        def _():
            pltpu.sync_copy(tbl.at[pl.ds(s * 256, 256)],
                            t1_hbm.at[pl.ds(s * 256, 256)])

    return scatter_k(h_new, cell2d, *init_args)


def _sc_gather(tabs, cell2d, iota2d):
    @functools.partial(
        pl.kernel,
        out_type=jax.ShapeDtypeStruct((N, HS), jnp.float32),
        mesh=_sc_mesh(),
        scratch_types=[
            pltpu.VMEM_SHARED((NCELL, HS), jnp.float32),
            pltpu.VMEM((2, HS), jnp.int32),
            pltpu.VMEM((_RPG, HS), jnp.int32),
            pltpu.VMEM((_NSLOT, HS, HS), jnp.float32),
        ] + [pltpu.SemaphoreType.DMA] * _NSLOT,
    )
    def gather_k(ta_hbm, tb_hbm, ca_hbm, cb_hbm, iota_hbm, o_hbm,
                 tbl, iv, idx_v, buf, *sems):
        c = lax.axis_index("c")
        s = lax.axis_index("s")
        pltpu.sync_copy(iota_hbm.at[pl.ds(2 * s, 2)], iv)

        def stage(own_hbm, other_hbm):
            for h in range(2):
                pltpu.async_copy(
                    other_hbm.at[pl.ds(s * 256 + h * HS, HS)], buf.at[h],
                    sems[h])
            pltpu.sync_copy(own_hbm.at[pl.ds(s * 256, 256)],
                            tbl.at[pl.ds(s * 256, 256)])
            for h in range(2):
                pltpu.make_async_copy(
                    other_hbm.at[pl.ds(s * 256 + h * HS, HS)], buf.at[h],
                    sems[h]).wait()
                pltpu.async_copy(buf.at[h], tbl.at[iv.at[h]], sems[h],
                                 add=True)
            for h in range(2):
                pltpu.make_async_copy(buf.at[h], tbl.at[iv.at[h]],
                                      sems[h]).wait()

        @pl.when(c == 0)
        def _():
            stage(ta_hbm, tb_hbm)

        @pl.when(c == 1)
        def _():
            stage(tb_hbm, ta_hbm)
        plsc.subcore_barrier()
        row0 = (c * 16 + s) * _RPG

        @pl.when(c == 0)
        def _():
            pltpu.sync_copy(ca_hbm.at[pl.ds(s * _RPG, _RPG)], idx_v)

        @pl.when(c == 1)
        def _():
            pltpu.sync_copy(cb_hbm.at[pl.ds(s * _RPG, _RPG)], idx_v)

        def gath(w, start):
            slot = w % _NSLOT
            args = (tbl.at[idx_v.at[w]], buf.at[slot], sems[slot])
            if start:
                pltpu.async_copy(*args)
            else:
                pltpu.make_async_copy(*args).wait()

        def wout(w, start):
            slot = w % _NSLOT
            args = (buf.at[slot], o_hbm.at[pl.ds((row0 + w) * HS, HS)],
                    sems[slot])
            if start:
                pltpu.async_copy(*args)
            else:
                pltpu.make_async_copy(*args).wait()

        for w in range(min(_NSLOT, _RPG)):
            gath(w, True)
        for w in range(_RPG):
            gath(w, False)
            wout(w, True)
            if w + _NSLOT < _RPG:
                wout(w, False)
                gath(w + _NSLOT, True)
        for w in range(max(0, _RPG - _NSLOT), _RPG):
            wout(w, False)

    return gather_k(tabs[0], tabs[1], cell2d[0], cell2d[1], iota2d)


def kernel(coords, hidden_state, cell_state, W_ih, W_hh, b_ih, b_hh):
    wih_t = W_ih.T.astype(jnp.bfloat16)
    whh_t = W_hh.T.astype(jnp.bfloat16)
    b_i = b_ih[None, :]
    b_h = b_hh[None, :]
    coords_t = coords.T.astype(jnp.bfloat16)
    xs2d = coords[:, 0].reshape(_NROW, HS)
    ys2d = coords[:, 1].reshape(_NROW, HS)
    hs, cds = [], []
    c_buf = None
    for k in range(_NCHUNK):
        hk, c_buf, cdk = _lstm_tc(k, coords_t, hidden_state, cell_state,
                                  xs2d, ys2d, wih_t, whh_t, b_i, b_h,
                                  c_donate=c_buf)
        hs.append(hk)
        cds.append(cdk)
    iota2d = jnp.arange(NCELL, dtype=jnp.int32).reshape(NCELL // HS, HS)
    tabs = None
    for k in range(_NCHUNK):
        tabs = _sc_scatter(hs[k], cds[k], inits=tabs)
    h_social = _sc_gather(tabs, cds, iota2d)
    return (h_social, c_buf)


# speedup vs baseline: 1.0468x; 1.0049x over previous
import functools
import jax
import jax.numpy as jnp
from jax import lax
from jax.experimental import pallas as pl
from jax.experimental.pallas import tpu as pltpu
from jax.experimental.pallas import tpu_sc as plsc

N = 65536
HS = 128
NG = 64
NCELL = NG * NG
X_MIN, X_MAX = -3.0, 3.0
Y_MIN, Y_MAX = -3.0, 3.0
DX = (X_MAX - X_MIN) / NG
DY = (Y_MAX - Y_MIN) / NG

_TC_B = 2048
_NROW = N // HS
_RB = _TC_B // HS


def _lstm_tc_body(x_ref, h_ref, c_ref, xs_ref, ys_ref, wih_ref, whh_ref,
                  bi_ref, bh_ref, hnew_ref, cnew_ref, cell_ref):
    xt = x_ref[...]
    h = h_ref[...].astype(jnp.bfloat16)
    c = c_ref[...]
    gates = (lax.dot_general(xt, wih_ref[...], (((0,), (0,)), ((), ())),
                             preferred_element_type=jnp.float32)
             + jnp.dot(h, whh_ref[...], preferred_element_type=jnp.float32)
             + (bi_ref[...] + bh_ref[...]))

    def sigmoid(z):
        return 0.5 * jnp.tanh(0.5 * z) + 0.5

    i = sigmoid(gates[:, 0:HS])
    f = sigmoid(gates[:, HS:2 * HS])
    g = jnp.tanh(gates[:, 2 * HS:3 * HS])
    o = sigmoid(gates[:, 3 * HS:4 * HS])
    c_new = f * c + i * g
    hnew_ref[...] = o * jnp.tanh(c_new)
    cnew_ref[...] = c_new
    xc = jnp.clip(xs_ref[...], X_MIN, X_MAX)
    yc = jnp.clip(ys_ref[...], Y_MIN, Y_MAX)
    xi = jnp.clip(jnp.floor((xc - X_MIN) / DX).astype(jnp.int32), 0, NG - 1)
    yi = jnp.clip(jnp.floor((yc - Y_MIN) / DY).astype(jnp.int32), 0, NG - 1)
    cell_ref[...] = xi * NG + yi


_NCHUNK = 2
_CB = N // _NCHUNK // _TC_B
_CROW = _NROW // _NCHUNK


def _lstm_tc_body2(x_ref, h_ref, c_ref, xs_ref, ys_ref, wih_ref, whh_ref,
                   bi_ref, bh_ref, cdest_ref, hnew_ref, cnew_ref, cell_ref):
    del cdest_ref
    _lstm_tc_body(x_ref, h_ref, c_ref, xs_ref, ys_ref, wih_ref, whh_ref,
                  bi_ref, bh_ref, hnew_ref, cnew_ref, cell_ref)


def _lstm_tc(off, nb, coords_t, h, c, xs2d, ys2d, wih_t, whh_t, b_i, b_h,
             c_donate=None, interpret=False):
    in_specs = [
        pl.BlockSpec((3, _TC_B), lambda i: (0, i + off)),
        pl.BlockSpec((_TC_B, HS), lambda i: (i + off, 0)),
        pl.BlockSpec((_TC_B, HS), lambda i: (i + off, 0)),
        pl.BlockSpec((_RB, HS), lambda i: (i + off, 0)),
        pl.BlockSpec((_RB, HS), lambda i: (i + off, 0)),
        pl.BlockSpec((3, 4 * HS), lambda i: (0, 0)),
        pl.BlockSpec((HS, 4 * HS), lambda i: (0, 0)),
        pl.BlockSpec((1, 4 * HS), lambda i: (0, 0)),
        pl.BlockSpec((1, 4 * HS), lambda i: (0, 0)),
    ]
    args = [coords_t, h, c, xs2d, ys2d, wih_t, whh_t, b_i, b_h]
    if c_donate is None:
        body = _lstm_tc_body
        aliases = {}
    else:
        body = _lstm_tc_body2
        in_specs = in_specs + [pl.BlockSpec((8, HS), lambda i: (0, 0))]
        args = args + [c_donate]
        aliases = {9: 1}
    return pl.pallas_call(
        body,
        grid=(nb,),
        in_specs=in_specs,
        out_specs=[
            pl.BlockSpec((_TC_B, HS), lambda i: (i, 0)),
            pl.BlockSpec((_TC_B, HS), lambda i: (i + off, 0)),
            pl.BlockSpec((_RB, HS), lambda i: (i, 0)),
        ],
        out_shape=[
            jax.ShapeDtypeStruct((nb * _TC_B, HS), jnp.float32),
            jax.ShapeDtypeStruct((N, HS), jnp.float32),
            jax.ShapeDtypeStruct((nb * _RB, HS), jnp.int32),
        ],
        input_output_aliases=aliases,
        interpret=interpret,
    )(*args)


@functools.cache
def _sc_mesh():
    return plsc.VectorSubcoreMesh(core_axis_name="c", subcore_axis_name="s",
                                  num_cores=2, num_subcores=16)
_RPC = _CROW // 2 // 16
_RPG = _NROW // 32


_NSLOT = 5


def _sc_scatter(h_new, cell2d, inits=None):
    init_args = [] if inits is None else list(inits)
    crow = cell2d.shape[0]
    rpc = crow // 32
    @functools.partial(
        pl.kernel,
        out_type=[jax.ShapeDtypeStruct((NCELL, HS), jnp.float32),
                  jax.ShapeDtypeStruct((NCELL, HS), jnp.float32)],
        mesh=_sc_mesh(),
        scratch_types=[
            pltpu.VMEM_SHARED((NCELL, HS), jnp.float32),
            pltpu.VMEM((rpc, HS), jnp.int32),
            pltpu.VMEM((_NSLOT, HS, HS), jnp.float32),
        ] + [pltpu.SemaphoreType.DMA] * _NSLOT,
    )
    def scatter_k(h_hbm, cell_hbm, *rest):
        if inits is None:
            (t0_hbm, t1_hbm, tbl, idx_v, hbuf), sems = rest[:5], rest[5:]
        else:
            (i0_hbm, i1_hbm, t0_hbm, t1_hbm, tbl, idx_v, hbuf) = rest[:7]
            sems = rest[7:]
        c = lax.axis_index("c")
        s = lax.axis_index("s")
        if inits is None:
            @pl.loop(0, HS)
            def _(r):
                for cb in range(HS // 16):
                    hbuf.at[0, r, pl.ds(cb * 16, 16)][...] = jnp.zeros(
                        (16,), jnp.float32)

            pltpu.sync_copy(hbuf.at[0], tbl.at[pl.ds(s * 256, HS)])
            pltpu.sync_copy(hbuf.at[0], tbl.at[pl.ds(s * 256 + HS, HS)])
        else:
            @pl.when(c == 0)
            def _():
                pltpu.sync_copy(i0_hbm.at[pl.ds(s * 256, 256)],
                                tbl.at[pl.ds(s * 256, 256)])

            @pl.when(c == 1)
            def _():
                pltpu.sync_copy(i1_hbm.at[pl.ds(s * 256, 256)],
                                tbl.at[pl.ds(s * 256, 256)])
        plsc.subcore_barrier()
        row0 = c * (crow // 2) + s * rpc
        pltpu.sync_copy(cell_hbm.at[pl.ds(row0, rpc)], idx_v)

        def load(w, start):
            slot = w % _NSLOT
            args = (h_hbm.at[pl.ds((row0 + w) * HS, HS)], hbuf.at[slot],
                    sems[slot])
            if start:
                pltpu.async_copy(*args)
            else:
                pltpu.make_async_copy(*args).wait()

        def scat(w, start):
            slot = w % _NSLOT
            args = (hbuf.at[slot], tbl.at[idx_v.at[w]], sems[slot])
            if start:
                pltpu.async_copy(*args, add=True)
            else:
                pltpu.make_async_copy(*args).wait()

        for w in range(min(_NSLOT, rpc)):
            load(w, True)
        for w in range(rpc):
            load(w, False)
            scat(w, True)
            if w + _NSLOT < rpc:
                scat(w, False)
                load(w + _NSLOT, True)
        for w in range(max(0, rpc - _NSLOT), rpc):
            scat(w, False)

        plsc.subcore_barrier()

        @pl.when(c == 0)
        def _():
            pltpu.sync_copy(tbl.at[pl.ds(s * 256, 256)],
                            t0_hbm.at[pl.ds(s * 256, 256)])

        @pl.when(c == 1)
        def _():
            pltpu.sync_copy(tbl.at[pl.ds(s * 256, 256)],
                            t1_hbm.at[pl.ds(s * 256, 256)])

    return scatter_k(h_new, cell2d, *init_args)


def _sc_gather(tabs, cell2d, iota2d):
    @functools.partial(
        pl.kernel,
        out_type=jax.ShapeDtypeStruct((N, HS), jnp.float32),
        mesh=_sc_mesh(),
        scratch_types=[
            pltpu.VMEM_SHARED((NCELL, HS), jnp.float32),
            pltpu.VMEM((2, HS), jnp.int32),
            pltpu.VMEM((_RPG, HS), jnp.int32),
            pltpu.VMEM((_NSLOT, HS, HS), jnp.float32),
        ] + [pltpu.SemaphoreType.DMA] * _NSLOT,
    )
    def gather_k(ta_hbm, tb_hbm, ca_hbm, cb_hbm, cc_hbm, iota_hbm, o_hbm,
                 tbl, iv, idx_v, buf, *sems):
        c = lax.axis_index("c")
        s = lax.axis_index("s")
        pltpu.sync_copy(iota_hbm.at[pl.ds(2 * s, 2)], iv)

        def stage(own_hbm, other_hbm):
            for h in range(2):
                pltpu.async_copy(
                    other_hbm.at[pl.ds(s * 256 + h * HS, HS)], buf.at[h],
                    sems[h])
            pltpu.sync_copy(own_hbm.at[pl.ds(s * 256, 256)],
                            tbl.at[pl.ds(s * 256, 256)])
            for h in range(2):
                pltpu.make_async_copy(
                    other_hbm.at[pl.ds(s * 256 + h * HS, HS)], buf.at[h],
                    sems[h]).wait()
                pltpu.async_copy(buf.at[h], tbl.at[iv.at[h]], sems[h],
                                 add=True)
            for h in range(2):
                pltpu.make_async_copy(buf.at[h], tbl.at[iv.at[h]],
                                      sems[h]).wait()

        @pl.when(c == 0)
        def _():
            stage(ta_hbm, tb_hbm)

        @pl.when(c == 1)
        def _():
            stage(tb_hbm, ta_hbm)
        plsc.subcore_barrier()
        row0 = (c * 16 + s) * _RPG

        @pl.when(c == 0)
        def _():
            pltpu.sync_copy(ca_hbm.at[pl.ds(s * _RPG, _RPG)], idx_v)

        @pl.when(jnp.logical_and(c == 1, s < 8))
        def _():
            pltpu.sync_copy(cb_hbm.at[pl.ds(s * _RPG, _RPG)], idx_v)

        @pl.when(jnp.logical_and(c == 1, s >= 8))
        def _():
            pltpu.sync_copy(cc_hbm.at[pl.ds((s - 8) * _RPG, _RPG)], idx_v)

        def gath(w, start):
            slot = w % _NSLOT
            args = (tbl.at[idx_v.at[w]], buf.at[slot], sems[slot])
            if start:
                pltpu.async_copy(*args)
            else:
                pltpu.make_async_copy(*args).wait()

        def wout(w, start):
            slot = w % _NSLOT
            args = (buf.at[slot], o_hbm.at[pl.ds((row0 + w) * HS, HS)],
                    sems[slot])
            if start:
                pltpu.async_copy(*args)
            else:
                pltpu.make_async_copy(*args).wait()

        for w in range(min(_NSLOT, _RPG)):
            gath(w, True)
        for w in range(_RPG):
            gath(w, False)
            wout(w, True)
            if w + _NSLOT < _RPG:
                wout(w, False)
                gath(w + _NSLOT, True)
        for w in range(max(0, _RPG - _NSLOT), _RPG):
            wout(w, False)

    return gather_k(tabs[0], tabs[1], cell2d[0], cell2d[1], cell2d[2], iota2d)


def kernel(coords, hidden_state, cell_state, W_ih, W_hh, b_ih, b_hh):
    wih_t = W_ih.T.astype(jnp.bfloat16)
    whh_t = W_hh.T.astype(jnp.bfloat16)
    b_i = b_ih[None, :]
    b_h = b_hh[None, :]
    coords_t = coords.T.astype(jnp.bfloat16)
    xs2d = coords[:, 0].reshape(_NROW, HS)
    ys2d = coords[:, 1].reshape(_NROW, HS)
    hs, cds = [], []
    c_buf = None
    for off, nb in ((0, 16), (16, 8), (24, 8)):
        hk, c_buf, cdk = _lstm_tc(off, nb, coords_t, hidden_state,
                                  cell_state, xs2d, ys2d, wih_t, whh_t,
                                  b_i, b_h, c_donate=c_buf)
        hs.append(hk)
        cds.append(cdk)
    iota2d = jnp.arange(NCELL, dtype=jnp.int32).reshape(NCELL // HS, HS)
    tabs = None
    for k in range(3):
        tabs = _sc_scatter(hs[k], cds[k], inits=tabs)
    h_social = _sc_gather(tabs, cds, iota2d)
    return (h_social, c_buf)


# speedup vs baseline: 1.0469x; 1.0000x over previous
import functools
import jax
import jax.numpy as jnp
from jax import lax
from jax.experimental import pallas as pl
from jax.experimental.pallas import tpu as pltpu
from jax.experimental.pallas import tpu_sc as plsc

N = 65536
HS = 128
NG = 64
NCELL = NG * NG
X_MIN, X_MAX = -3.0, 3.0
Y_MIN, Y_MAX = -3.0, 3.0
DX = (X_MAX - X_MIN) / NG
DY = (Y_MAX - Y_MIN) / NG

_TC_B = 2048
_NROW = N // HS
_RB = _TC_B // HS


def _lstm_tc_body(x_ref, h_ref, c_ref, xs_ref, ys_ref, wih_ref, whh_ref,
                  bi_ref, bh_ref, hnew_ref, cnew_ref, cell_ref):
    xt = x_ref[...]
    h = h_ref[...].astype(jnp.bfloat16)
    c = c_ref[...]
    gates = (lax.dot_general(xt, wih_ref[...], (((0,), (0,)), ((), ())),
                             preferred_element_type=jnp.float32)
             + jnp.dot(h, whh_ref[...], preferred_element_type=jnp.float32)
             + (bi_ref[...] + bh_ref[...]))

    def sigmoid(z):
        return 0.5 * jnp.tanh(0.5 * z) + 0.5

    i = sigmoid(gates[:, 0:HS])
    f = sigmoid(gates[:, HS:2 * HS])
    g = jnp.tanh(gates[:, 2 * HS:3 * HS])
    o = sigmoid(gates[:, 3 * HS:4 * HS])
    c_new = f * c + i * g
    hnew_ref[...] = o * jnp.tanh(c_new)
    cnew_ref[...] = c_new
    xc = jnp.clip(xs_ref[...], X_MIN, X_MAX)
    yc = jnp.clip(ys_ref[...], Y_MIN, Y_MAX)
    xi = jnp.clip(jnp.floor((xc - X_MIN) / DX).astype(jnp.int32), 0, NG - 1)
    yi = jnp.clip(jnp.floor((yc - Y_MIN) / DY).astype(jnp.int32), 0, NG - 1)
    cell_ref[...] = xi * NG + yi


def _lstm_tc_body2(x_ref, h_ref, c_ref, xs_ref, ys_ref, wih_ref, whh_ref,
                   bi_ref, bh_ref, cdest_ref, hnew_ref, cnew_ref, cell_ref):
    del cdest_ref
    _lstm_tc_body(x_ref, h_ref, c_ref, xs_ref, ys_ref, wih_ref, whh_ref,
                  bi_ref, bh_ref, hnew_ref, cnew_ref, cell_ref)


def _lstm_tc(off, nb, coords_t, h, c, xs2d, ys2d, wih_t, whh_t, b_i, b_h,
             c_donate=None, interpret=False):
    in_specs = [
        pl.BlockSpec((3, _TC_B), lambda i: (0, i + off)),
        pl.BlockSpec((_TC_B, HS), lambda i: (i + off, 0)),
        pl.BlockSpec((_TC_B, HS), lambda i: (i + off, 0)),
        pl.BlockSpec((_RB, HS), lambda i: (i + off, 0)),
        pl.BlockSpec((_RB, HS), lambda i: (i + off, 0)),
        pl.BlockSpec((3, 4 * HS), lambda i: (0, 0)),
        pl.BlockSpec((HS, 4 * HS), lambda i: (0, 0)),
        pl.BlockSpec((1, 4 * HS), lambda i: (0, 0)),
        pl.BlockSpec((1, 4 * HS), lambda i: (0, 0)),
    ]
    args = [coords_t, h, c, xs2d, ys2d, wih_t, whh_t, b_i, b_h]
    if c_donate is None:
        body = _lstm_tc_body
        aliases = {}
    else:
        body = _lstm_tc_body2
        in_specs = in_specs + [pl.BlockSpec((8, HS), lambda i: (0, 0))]
        args = args + [c_donate]
        aliases = {9: 1}
    return pl.pallas_call(
        body,
        grid=(nb,),
        in_specs=in_specs,
        out_specs=[
            pl.BlockSpec((_TC_B, HS), lambda i: (i, 0)),
            pl.BlockSpec((_TC_B, HS), lambda i: (i + off, 0)),
            pl.BlockSpec((_RB, HS), lambda i: (i, 0)),
        ],
        out_shape=[
            jax.ShapeDtypeStruct((nb * _TC_B, HS), jnp.float32),
            jax.ShapeDtypeStruct((N, HS), jnp.float32),
            jax.ShapeDtypeStruct((nb * _RB, HS), jnp.int32),
        ],
        input_output_aliases=aliases,
        interpret=interpret,
    )(*args)


@functools.cache
def _sc_mesh():
    return plsc.VectorSubcoreMesh(core_axis_name="c", subcore_axis_name="s",
                                  num_cores=2, num_subcores=16)
_RPG = _NROW // 32


_NSLOT = 5


def _sc_scatter(h_new, cell2d, inits=None):
    init_args = [] if inits is None else list(inits)
    crow = cell2d.shape[0]
    rpc = crow // 32
    @functools.partial(
        pl.kernel,
        out_type=[jax.ShapeDtypeStruct((NCELL, HS), jnp.float32),
                  jax.ShapeDtypeStruct((NCELL, HS), jnp.float32)],
        mesh=_sc_mesh(),
        scratch_types=[
            pltpu.VMEM_SHARED((NCELL, HS), jnp.float32),
            pltpu.VMEM((rpc, HS), jnp.int32),
            pltpu.VMEM((_NSLOT, HS, HS), jnp.float32),
        ] + [pltpu.SemaphoreType.DMA] * _NSLOT,
    )
    def scatter_k(h_hbm, cell_hbm, *rest):
        if inits is None:
            (t0_hbm, t1_hbm, tbl, idx_v, hbuf), sems = rest[:5], rest[5:]
        else:
            (i0_hbm, i1_hbm, t0_hbm, t1_hbm, tbl, idx_v, hbuf) = rest[:7]
            sems = rest[7:]
        c = lax.axis_index("c")
        s = lax.axis_index("s")
        if inits is None:
            @pl.loop(0, HS)
            def _(r):
                for cb in range(HS // 16):
                    hbuf.at[0, r, pl.ds(cb * 16, 16)][...] = jnp.zeros(
                        (16,), jnp.float32)

            pltpu.sync_copy(hbuf.at[0], tbl.at[pl.ds(s * 256, HS)])
            pltpu.sync_copy(hbuf.at[0], tbl.at[pl.ds(s * 256 + HS, HS)])
        else:
            @pl.when(c == 0)
            def _():
                pltpu.sync_copy(i0_hbm.at[pl.ds(s * 256, 256)],
                                tbl.at[pl.ds(s * 256, 256)])

            @pl.when(c == 1)
            def _():
                pltpu.sync_copy(i1_hbm.at[pl.ds(s * 256, 256)],
                                tbl.at[pl.ds(s * 256, 256)])
        plsc.subcore_barrier()
        row0 = c * (crow // 2) + s * rpc
        pltpu.sync_copy(cell_hbm.at[pl.ds(row0, rpc)], idx_v)

        def load(w, start):
            slot = w % _NSLOT
            args = (h_hbm.at[pl.ds((row0 + w) * HS, HS)], hbuf.at[slot],
                    sems[slot])
            if start:
                pltpu.async_copy(*args)
            else:
                pltpu.make_async_copy(*args).wait()

        def scat(w, start):
            slot = w % _NSLOT
            args = (hbuf.at[slot], tbl.at[idx_v.at[w]], sems[slot])
            if start:
                pltpu.async_copy(*args, add=True)
            else:
                pltpu.make_async_copy(*args).wait()

        for w in range(min(_NSLOT, rpc)):
            load(w, True)
        for w in range(rpc):
            load(w, False)
            scat(w, True)
            if w + _NSLOT < rpc:
                scat(w, False)
                load(w + _NSLOT, True)
        for w in range(max(0, rpc - _NSLOT), rpc):
            scat(w, False)

        plsc.subcore_barrier()

        @pl.when(c == 0)
        def _():
            pltpu.sync_copy(tbl.at[pl.ds(s * 256, 256)],
                            t0_hbm.at[pl.ds(s * 256, 256)])

        @pl.when(c == 1)
        def _():
            pltpu.sync_copy(tbl.at[pl.ds(s * 256, 256)],
                            t1_hbm.at[pl.ds(s * 256, 256)])

    return scatter_k(h_new, cell2d, *init_args)


def _sc_gather(tabs, cell2d, iota2d):
    @functools.partial(
        pl.kernel,
        out_type=jax.ShapeDtypeStruct((N, HS), jnp.float32),
        mesh=_sc_mesh(),
        scratch_types=[
            pltpu.VMEM_SHARED((NCELL, HS), jnp.float32),
            pltpu.VMEM((2, HS), jnp.int32),
            pltpu.VMEM((_RPG, HS), jnp.int32),
            pltpu.VMEM((_NSLOT, HS, HS), jnp.float32),
        ] + [pltpu.SemaphoreType.DMA] * _NSLOT,
    )
    def gather_k(ta_hbm, tb_hbm, ca_hbm, cb_hbm, cc_hbm, iota_hbm, o_hbm,
                 tbl, iv, idx_v, buf, *sems):
        c = lax.axis_index("c")
        s = lax.axis_index("s")
        pltpu.sync_copy(iota_hbm.at[pl.ds(2 * s, 2)], iv)

        def stage(own_hbm, other_hbm):
            for h in range(2):
                pltpu.async_copy(
                    other_hbm.at[pl.ds(s * 256 + h * HS, HS)], buf.at[h],
                    sems[h])
            pltpu.sync_copy(own_hbm.at[pl.ds(s * 256, 256)],
                            tbl.at[pl.ds(s * 256, 256)])
            for h in range(2):
                pltpu.make_async_copy(
                    other_hbm.at[pl.ds(s * 256 + h * HS, HS)], buf.at[h],
                    sems[h]).wait()
                pltpu.async_copy(buf.at[h], tbl.at[iv.at[h]], sems[h],
                                 add=True)
            for h in range(2):
                pltpu.make_async_copy(buf.at[h], tbl.at[iv.at[h]],
                                      sems[h]).wait()

        @pl.when(c == 0)
        def _():
            stage(ta_hbm, tb_hbm)

        @pl.when(c == 1)
        def _():
            stage(tb_hbm, ta_hbm)
        plsc.subcore_barrier()
        row0 = (c * 16 + s) * _RPG

        @pl.when(c == 0)
        def _():
            pltpu.sync_copy(ca_hbm.at[pl.ds(s * _RPG, _RPG)], idx_v)

        @pl.when(jnp.logical_and(c == 1, s < 8))
        def _():
            pltpu.sync_copy(cb_hbm.at[pl.ds(s * _RPG, _RPG)], idx_v)

        @pl.when(jnp.logical_and(c == 1, s >= 8))
        def _():
            pltpu.sync_copy(cc_hbm.at[pl.ds((s - 8) * _RPG, _RPG)], idx_v)

        def gath(w, start):
            slot = w % _NSLOT
            args = (tbl.at[idx_v.at[w]], buf.at[slot], sems[slot])
            if start:
                pltpu.async_copy(*args)
            else:
                pltpu.make_async_copy(*args).wait()

        def wout(w, start):
            slot = w % _NSLOT
            args = (buf.at[slot], o_hbm.at[pl.ds((row0 + w) * HS, HS)],
                    sems[slot])
            if start:
                pltpu.async_copy(*args)
            else:
                pltpu.make_async_copy(*args).wait()

        for w in range(min(_NSLOT, _RPG)):
            gath(w, True)
        for w in range(_RPG):
            gath(w, False)
            wout(w, True)
            if w + _NSLOT < _RPG:
                wout(w, False)
                gath(w + _NSLOT, True)
        for w in range(max(0, _RPG - _NSLOT), _RPG):
            wout(w, False)

    return gather_k(tabs[0], tabs[1], cell2d[0], cell2d[1], cell2d[2], iota2d)


def kernel(coords, hidden_state, cell_state, W_ih, W_hh, b_ih, b_hh):
    wih_t = W_ih.T.astype(jnp.bfloat16)
    whh_t = W_hh.T.astype(jnp.bfloat16)
    b_i = b_ih[None, :]
    b_h = b_hh[None, :]
    coords_t = coords.T.astype(jnp.bfloat16)
    xs2d = coords[:, 0].reshape(_NROW, HS)
    ys2d = coords[:, 1].reshape(_NROW, HS)
    hs, cds = [], []
    c_buf = None
    for off, nb in ((0, 16), (16, 8), (24, 8)):
        hk, c_buf, cdk = _lstm_tc(off, nb, coords_t, hidden_state,
                                  cell_state, xs2d, ys2d, wih_t, whh_t,
                                  b_i, b_h, c_donate=c_buf)
        hs.append(hk)
        cds.append(cdk)
    iota2d = jnp.arange(NCELL, dtype=jnp.int32).reshape(NCELL // HS, HS)
    tabs = None
    for k in range(3):
        tabs = _sc_scatter(hs[k], cds[k], inits=tabs)
    h_social = _sc_gather(tabs, cds, iota2d)
    return (h_social, c_buf)
